# Initial kernel scaffold; baseline (speedup 1.0000x reference)
import jax
import jax.numpy as jnp
from jax.experimental import pallas as pl


def kernel(x, edge_index, edge_types, graph_ids, W_et, b_et, W_ih, W_hh, b_ih, b_hh, W1, b1, W2, b2):
    raise NotImplementedError("write your pallas kernel here")


# speedup vs baseline: 12.0313x; 12.0313x over previous
import functools

import jax
import jax.numpy as jnp
from jax import lax
from jax.experimental import pallas as pl
from jax.experimental.pallas import tpu as pltpu
from jax.experimental.pallas import tpu_sc as plsc

N = 10000
E = 320000
D = 128
ET = 4
STEPS = 8
G = 16
HID = 256

NPAD = 10240
ROWS_PER_TILE = NPAD // 16
CH = 128
NCHUNK = 79
EPT = CH * NCHUNK
EPAD = EPT * 32
HALF = EPAD // 2

BN = 400
NB = N // BN

_HIGH = lax.Precision.HIGHEST


def _dot(a, b, dims):
    return lax.dot_general(a, b, (dims, ((), ())), precision=_HIGH,
                           preferred_element_type=jnp.float32)


def _tc1_body(h_ref, w_ref, b_ref, ht_ref, gh_ref):
    acc = _dot(h_ref[...], w_ref[...], ((1,), (0,))) + b_ref[...]
    ht_ref[...] = acc[:, :ET * D]
    gh_ref[...] = acc[:, ET * D:]


def _tc1(h, wbig, bbig):
    return pl.pallas_call(
        _tc1_body,
        grid=(NB,),
        in_specs=[
            pl.BlockSpec((BN, D), lambda i: (i, 0)),
            pl.BlockSpec((D, (ET + 3) * D), lambda i: (0, 0)),
            pl.BlockSpec((1, (ET + 3) * D), lambda i: (0, 0)),
        ],
        out_specs=[
            pl.BlockSpec((BN, ET * D), lambda i: (i, 0)),
            pl.BlockSpec((BN, 3 * D), lambda i: (i, 0)),
        ],
        out_shape=[
            jax.ShapeDtypeStruct((N, ET * D), jnp.float32),
            jax.ShapeDtypeStruct((N, 3 * D), jnp.float32),
        ],
    )(h, wbig, bbig)


@functools.partial(
    pl.kernel,
    out_type=jax.ShapeDtypeStruct((2, NPAD, D), jnp.float32),
    mesh=plsc.VectorSubcoreMesh(core_axis_name="c", subcore_axis_name="s"),
    scratch_types=[
        pltpu.VMEM_SHARED((NPAD, D), jnp.float32),
        pltpu.VMEM((CH,), jnp.int32),
        pltpu.VMEM((CH,), jnp.int32),
        pltpu.VMEM((CH, D), jnp.float32),
    ],
)
def _sc_agg(ht_ref, gidx_ref, dst_ref, out_ref, acc_shr, idx_v, dst_v, rows_v):
    c = lax.axis_index("c")
    s = lax.axis_index("s")

    def _zrow(i, carry):
        for j in range(D // 16):
            rows_v[i, pl.ds(j * 16, 16)] = jnp.zeros((16,), jnp.float32)
        return carry

    lax.fori_loop(0, CH, _zrow, 0)
    row0 = pl.multiple_of(s * ROWS_PER_TILE, 8)
    for k in range(ROWS_PER_TILE // CH):
        pltpu.sync_copy(rows_v, acc_shr.at[pl.ds(row0 + k * CH, CH)])
    plsc.subcore_barrier()

    ebase = c * HALF + s * EPT

    def _chunk(k, carry):
        off = pl.multiple_of(ebase + k * CH, 8)
        pltpu.sync_copy(gidx_ref.at[pl.ds(off, CH)], idx_v)
        pltpu.sync_copy(dst_ref.at[pl.ds(off, CH)], dst_v)
        pltpu.sync_copy(ht_ref.at[idx_v], rows_v)
        pltpu.sync_copy(rows_v, acc_shr.at[dst_v], add=True)
        return carry

    lax.fori_loop(0, NCHUNK, _chunk, 0)
    plsc.subcore_barrier()
    pltpu.sync_copy(acc_shr.at[pl.ds(row0, ROWS_PER_TILE)],
                    out_ref.at[c, pl.ds(row0, ROWS_PER_TILE)])


def _gru_body(a_ref, gh_ref, h_ref, w_ref, b_ref, out_ref):
    a = a_ref[0] + a_ref[1]
    gi = _dot(a, w_ref[...], ((1,), (0,))) + b_ref[...]
    gh = gh_ref[...]
    h = h_ref[...]
    r = jax.nn.sigmoid(gi[:, :D] + gh[:, :D])
    z = jax.nn.sigmoid(gi[:, D:2 * D] + gh[:, D:2 * D])
    n = jnp.tanh(gi[:, 2 * D:] + r * gh[:, 2 * D:])
    out_ref[...] = (1.0 - z) * n + z * h


def _gru(a2, gh, h, wih_t, bih):
    return pl.pallas_call(
        _gru_body,
        grid=(NB,),
        in_specs=[
            pl.BlockSpec((2, BN, D), lambda i: (0, i, 0)),
            pl.BlockSpec((BN, 3 * D), lambda i: (i, 0)),
            pl.BlockSpec((BN, D), lambda i: (i, 0)),
            pl.BlockSpec((D, 3 * D), lambda i: (0, 0)),
            pl.BlockSpec((1, 3 * D), lambda i: (0, 0)),
        ],
        out_specs=pl.BlockSpec((BN, D), lambda i: (i, 0)),
        out_shape=jax.ShapeDtypeStruct((N, D), jnp.float32),
    )(a2, gh, h, wih_t, bih)


def _pool_body(h_ref, gid_ref, w1_ref, b1_ref, w2_ref, b2_ref, out_ref,
               acc_ref, cnt_ref):
    i = pl.program_id(0)

    @pl.when(i == 0)
    def _():
        acc_ref[...] = jnp.zeros((G, D), jnp.float32)
        cnt_ref[...] = jnp.zeros((G, D), jnp.float32)

    g = gid_ref[0, 0, :]
    onehot = (g[:, None] == lax.broadcasted_iota(jnp.int32, (1, G), 1)
              ).astype(jnp.float32)
    acc_ref[...] += _dot(onehot, h_ref[...], ((0,), (0,)))
    cnt_ref[...] += jnp.broadcast_to(jnp.sum(onehot, axis=0)[:, None], (G, D))

    @pl.when(i == NB - 1)
    def _():
        pooled = acc_ref[...] / jnp.maximum(cnt_ref[...], 1.0)
        hid = jax.nn.relu(_dot(pooled, w1_ref[...], ((1,), (1,))) + b1_ref[...])
        logits = jnp.sum(hid * w2_ref[...], axis=1)[:, None] + b2_ref[...]
        out_ref[...] = jnp.broadcast_to(jax.nn.sigmoid(logits), (G, D))


def _pool(h, gid3, w1, b1, w2, b2):
    return pl.pallas_call(
        _pool_body,
        grid=(NB,),
        in_specs=[
            pl.BlockSpec((BN, D), lambda i: (i, 0)),
            pl.BlockSpec((1, 1, BN), lambda i: (i, 0, 0)),
            pl.BlockSpec((HID, D), lambda i: (0, 0)),
            pl.BlockSpec((1, HID), lambda i: (0, 0)),
            pl.BlockSpec((1, HID), lambda i: (0, 0)),
            pl.BlockSpec((1, 1), lambda i: (0, 0)),
        ],
        out_specs=pl.BlockSpec((G, D), lambda i: (0, 0)),
        out_shape=jax.ShapeDtypeStruct((G, D), jnp.float32),
        scratch_shapes=[
            pltpu.VMEM((G, D), jnp.float32),
            pltpu.VMEM((G, D), jnp.float32),
        ],
    )(h, gid3, w1, b1, w2, b2)


def kernel(x, edge_index, edge_types, graph_ids, W_et, b_et, W_ih, W_hh,
           b_ih, b_hh, W1, b1, W2, b2):
    src = edge_index[0]
    dst = edge_index[1]
    gidx = src * ET + edge_types
    pad = EPAD - E
    gidx_p = jnp.concatenate([gidx, jnp.zeros((pad,), jnp.int32)])
    dst_p = jnp.concatenate([dst, jnp.full((pad,), N, jnp.int32)])

    wcat = jnp.concatenate([W_et[t].T for t in range(ET)], axis=1)
    bcat = jnp.concatenate([b_et[t] for t in range(ET)])
    wbig = jnp.concatenate([wcat, W_hh.T], axis=1)
    bbig = jnp.concatenate([bcat, b_hh]).reshape(1, -1)
    wih_t = W_ih.T
    bih = b_ih.reshape(1, -1)
    gid3 = graph_ids.reshape(NB, 1, BN)

    h = x
    for _ in range(STEPS):
        ht, gh = _tc1(h, wbig, bbig)
        ht2 = ht.reshape(N * ET, D)
        a2 = _sc_agg(ht2, gidx_p, dst_p)
        h = _gru(a2, gh, h, wih_t, bih)

    out = _pool(h, gid3, W1, b1.reshape(1, -1), W2, b2.reshape(1, 1))
    return out[:, :1]
